# baseline (speedup 1.0000x reference)
import math

import jax
import jax.numpy as jnp
from jax import lax
from jax.experimental import pallas as pl
from jax.experimental.pallas import tpu as pltpu
from jax.experimental.pallas import tpu_sc as plsc

N = 10000
E = 320000
D = 128
OUT = 64
NUM_LAYERS = 2
ALPHA = 0.1
THETA = 0.5

B = 80
NC, NS = 2, 16
NW = NC * NS
G = 5
KOB = 25
CH = G * KOB
NPT = 640
NPAD = NS * NPT


def _seg_sum_kernel(d, deg_mode):
    mesh = plsc.VectorSubcoreMesh(core_axis_name="c", subcore_axis_name="s")
    grp = d // 16

    def body(*refs):
        if deg_mode:
            (col_hbm, w_hbm, z_hbm, out_hbm,
             col_v, wr0, wr1, msg0, msg1, agg_sh,
             ws0, ws1, ss0, ss1) = refs
        else:
            (hp_hbm, row_hbm, col_hbm, w_hbm, z_hbm, out_hbm,
             row_v, col_v, wr0, wr1, msg0, msg1, agg_sh,
             ws0, ws1, ss0, ss1, gs0, gs1) = refs
            gs = [gs0, gs1]
        wr = [wr0, wr1]
        msg = [msg0, msg1]
        ws = [ws0, ws1]
        ss = [ss0, ss1]
        cid = lax.axis_index("c")
        sid = lax.axis_index("s")
        g = cid * NS + sid

        def drain(p):
            pltpu.make_async_copy(
                msg[p], agg_sh.at[col_v.at[0]], ss[p]).wait()

        pltpu.sync_copy(z_hbm, agg_sh.at[pl.ds(sid * NPT, NPT)])
        plsc.subcore_barrier()

        def block(ko, c):
            if not deg_mode:
                pltpu.sync_copy(row_hbm.at[g, ko], row_v)
            pltpu.sync_copy(col_hbm.at[g, ko], col_v)
            pltpu.async_copy(w_hbm.at[g, ko * G], wr[0], ws[0])
            if not deg_mode:
                pltpu.async_copy(hp_hbm.at[row_v.at[0]], msg[0], gs[0])

            for ki in range(G):
                p = ki % 2
                q = 1 - p
                k = ko * G + ki
                pltpu.make_async_copy(
                    w_hbm.at[g, k], wr[p], ws[p]).wait()
                if not deg_mode:
                    pltpu.make_async_copy(
                        hp_hbm.at[row_v.at[ki]], msg[p], gs[p]).wait()
                elif ki >= 2:
                    drain(p)
                if ki < G - 1:
                    if not deg_mode:
                        if ki >= 1:
                            drain(q)
                        pltpu.async_copy(
                            hp_hbm.at[row_v.at[ki + 1]], msg[q], gs[q])
                    pltpu.async_copy(w_hbm.at[g, k + 1], wr[q], ws[q])

                @plsc.parallel_loop(0, B, unroll=4)
                def edge(e, _p=p):
                    wv = wr[_p][e, :]
                    for j in range(grp):
                        if deg_mode:
                            msg[_p][e, pl.ds(16 * j, 16)] = wv
                        else:
                            msg[_p][e, pl.ds(16 * j, 16)] = (
                                msg[_p][e, pl.ds(16 * j, 16)] * wv)
                pltpu.async_copy(
                    msg[p], agg_sh.at[col_v.at[ki]], ss[p], add=True)
            drain((G - 2) % 2)
            drain((G - 1) % 2)
            return c
        lax.fori_loop(0, KOB, block, 0)
        plsc.subcore_barrier()

        base = cid * NPAD + sid * NPT
        pltpu.sync_copy(agg_sh.at[pl.ds(sid * NPT, NPT)],
                        out_hbm.at[pl.ds(base, NPT)])

    scratch = []
    if not deg_mode:
        scratch.append(pltpu.VMEM((G, B), jnp.int32))
    scratch.append(pltpu.VMEM((G, B), jnp.int32))
    scratch += [pltpu.VMEM((B, 16), jnp.float32)] * 2
    scratch += [pltpu.VMEM((B, d), jnp.float32)] * 2
    scratch.append(pltpu.VMEM_SHARED((NPAD, d), jnp.float32))
    nsem = 4 if deg_mode else 6
    scratch += [pltpu.SemaphoreType.DMA] * nsem
    return pl.kernel(
        body,
        out_type=jax.ShapeDtypeStruct((NC * NPAD, d), jnp.float32),
        mesh=mesh,
        scratch_types=scratch,
    )


def _prep_call(degparts, x):
    R = 2000

    def body(dp_ref, x_ref, dis_ref, hp_ref):
        deg = dp_ref[0, :, :16] + dp_ref[1, :, :16] + 1.0
        dis = lax.rsqrt(deg)
        dis_ref[...] = dis
        hp_ref[...] = x_ref[...] * dis[:, 0:1]

    return pl.pallas_call(
        body,
        grid=(N // R,),
        in_specs=[
            pl.BlockSpec((2, R, D), lambda i: (0, i, 0)),
            pl.BlockSpec((R, D), lambda i: (i, 0)),
        ],
        out_specs=[
            pl.BlockSpec((R, 16), lambda i: (i, 0)),
            pl.BlockSpec((R, D), lambda i: (i, 0)),
        ],
        out_shape=[
            jax.ShapeDtypeStruct((N, 16), jnp.float32),
            jax.ShapeDtypeStruct((N, D), jnp.float32),
        ],
    )(degparts, x)


def _dense_call(sparts, hp, x, dis16, w, beta, lin_wT=None, lin_bR=None):
    R = 2000
    last = lin_wT is not None

    def body(sp_ref, hp_ref, x_ref, dis_ref, w_ref, *rest):
        d1 = dis_ref[:, 0:1]
        agg = d1 * (sp_ref[0] + sp_ref[1] + hp_ref[...])
        agg = (1.0 - ALPHA) * agg + ALPHA * x_ref[...]
        out = (1.0 - beta) * agg + beta * jnp.dot(
            agg, w_ref[...], preferred_element_type=jnp.float32,
            precision=lax.Precision.HIGHEST)
        h = jnp.maximum(out, 0.0)
        if last:
            lw_ref, lb_ref, o_ref = rest
            o_ref[...] = jnp.dot(
                h, lw_ref[...], preferred_element_type=jnp.float32,
                precision=lax.Precision.HIGHEST) + lb_ref[...]
        else:
            (o_ref,) = rest
            o_ref[...] = d1 * h

    in_specs = [
        pl.BlockSpec((2, R, D), lambda i: (0, i, 0)),
        pl.BlockSpec((R, D), lambda i: (i, 0)),
        pl.BlockSpec((R, D), lambda i: (i, 0)),
        pl.BlockSpec((R, 16), lambda i: (i, 0)),
        pl.BlockSpec((D, D), lambda i: (0, 0)),
    ]
    args = [sparts, hp, x, dis16, w]
    if last:
        in_specs += [
            pl.BlockSpec((D, OUT), lambda i: (0, 0)),
            pl.BlockSpec((1, OUT), lambda i: (0, 0)),
        ]
        args += [lin_wT, lin_bR]
        out_spec = pl.BlockSpec((R, OUT), lambda i: (i, 0))
        out_shape = jax.ShapeDtypeStruct((N, OUT), jnp.float32)
    else:
        out_spec = pl.BlockSpec((R, D), lambda i: (i, 0))
        out_shape = jax.ShapeDtypeStruct((N, D), jnp.float32)

    return pl.pallas_call(
        body,
        grid=(N // R,),
        in_specs=in_specs,
        out_specs=out_spec,
        out_shape=out_shape,
    )(*args)


def kernel(x, edge_index, edge_weights, conv_w, lin_w, lin_b):
    row4 = edge_index[0].reshape(NW, KOB, G, B)
    col4 = edge_index[1].reshape(NW, KOB, G, B)
    wrep = jnp.broadcast_to(edge_weights[:, None], (E, 16))
    w4d = wrep.reshape(NW, CH, B, 16)
    z128 = jnp.zeros((NPT, D), jnp.float32)

    degparts = _seg_sum_kernel(D, True)(col4, w4d, z128).reshape(2, NPAD, D)
    dis16, hp = _prep_call(degparts, x)

    lin_wT = lin_w.T
    lin_bR = lin_b.reshape(1, OUT)

    seg = _seg_sum_kernel(D, False)
    for l in range(NUM_LAYERS):
        sparts = seg(hp, row4, col4, w4d, z128).reshape(2, NPAD, D)
        beta = math.log(THETA / (l + 1) + 1.0)
        if l < NUM_LAYERS - 1:
            hp = _dense_call(sparts, hp, x, dis16, conv_w[l], beta)
        else:
            out = _dense_call(sparts, hp, x, dis16, conv_w[l], beta,
                              lin_wT, lin_bR)
    return out


# speedup vs baseline: 1.1270x; 1.1270x over previous
import math

import jax
import jax.numpy as jnp
from jax import lax
from jax.experimental import pallas as pl
from jax.experimental.pallas import tpu as pltpu
from jax.experimental.pallas import tpu_sc as plsc

N = 10000
E = 320000
D = 128
OUT = 64
NUM_LAYERS = 2
ALPHA = 0.1
THETA = 0.5

B = 80
NC, NS = 2, 16
NW = NC * NS
G = 25
KOB = 5
CH = G * KOB
NPT = 640
NPAD = NS * NPT


def _seg_sum_kernel(d, deg_mode):
    mesh = plsc.VectorSubcoreMesh(core_axis_name="c", subcore_axis_name="s")
    grp = d // 16

    def body(*refs):
        if deg_mode:
            (col_hbm, w_hbm, z_hbm, out_hbm,
             col_v, wr0, wr1, msg0, msg1, agg_sh,
             ws0, ws1, ss0, ss1) = refs
        else:
            (hp_hbm, row_hbm, col_hbm, w_hbm, z_hbm, out_hbm,
             row_v, col_v, wr0, wr1, msg0, msg1, agg_sh,
             ws0, ws1, ss0, ss1, gs0, gs1) = refs
            gs = [gs0, gs1]
        wr = [wr0, wr1]
        msg = [msg0, msg1]
        ws = [ws0, ws1]
        ss = [ss0, ss1]
        cid = lax.axis_index("c")
        sid = lax.axis_index("s")
        g = cid * NS + sid

        def drain(p):
            pltpu.make_async_copy(
                msg[p], agg_sh.at[col_v.at[0]], ss[p]).wait()

        pltpu.sync_copy(z_hbm, agg_sh.at[pl.ds(sid * NPT, NPT)])
        plsc.subcore_barrier()

        def block(ko, c):
            if not deg_mode:
                pltpu.sync_copy(row_hbm.at[g, ko], row_v)
            pltpu.sync_copy(col_hbm.at[g, ko], col_v)
            pltpu.async_copy(w_hbm.at[g, ko * G], wr[0], ws[0])
            if not deg_mode:
                pltpu.async_copy(hp_hbm.at[row_v.at[0]], msg[0], gs[0])

            for ki in range(G):
                p = ki % 2
                q = 1 - p
                k = ko * G + ki
                pltpu.make_async_copy(
                    w_hbm.at[g, k], wr[p], ws[p]).wait()
                if not deg_mode:
                    pltpu.make_async_copy(
                        hp_hbm.at[row_v.at[ki]], msg[p], gs[p]).wait()
                elif ki >= 2:
                    drain(p)
                if ki < G - 1:
                    if not deg_mode:
                        if ki >= 1:
                            drain(q)
                        pltpu.async_copy(
                            hp_hbm.at[row_v.at[ki + 1]], msg[q], gs[q])
                    pltpu.async_copy(w_hbm.at[g, k + 1], wr[q], ws[q])

                @plsc.parallel_loop(0, B, unroll=4)
                def edge(e, _p=p):
                    wv = wr[_p][e, :]
                    for j in range(grp):
                        if deg_mode:
                            msg[_p][e, pl.ds(16 * j, 16)] = wv
                        else:
                            msg[_p][e, pl.ds(16 * j, 16)] = (
                                msg[_p][e, pl.ds(16 * j, 16)] * wv)
                pltpu.async_copy(
                    msg[p], agg_sh.at[col_v.at[ki]], ss[p], add=True)
            drain((G - 2) % 2)
            drain((G - 1) % 2)
            return c
        lax.fori_loop(0, KOB, block, 0)
        plsc.subcore_barrier()

        base = cid * NPAD + sid * NPT
        pltpu.sync_copy(agg_sh.at[pl.ds(sid * NPT, NPT)],
                        out_hbm.at[pl.ds(base, NPT)])

    scratch = []
    if not deg_mode:
        scratch.append(pltpu.VMEM((G, B), jnp.int32))
    scratch.append(pltpu.VMEM((G, B), jnp.int32))
    scratch += [pltpu.VMEM((B, 16), jnp.float32)] * 2
    scratch += [pltpu.VMEM((B, d), jnp.float32)] * 2
    scratch.append(pltpu.VMEM_SHARED((NPAD, d), jnp.float32))
    nsem = 4 if deg_mode else 6
    scratch += [pltpu.SemaphoreType.DMA] * nsem
    return pl.kernel(
        body,
        out_type=jax.ShapeDtypeStruct((NC * NPAD, d), jnp.float32),
        mesh=mesh,
        scratch_types=scratch,
    )


def _prep_call(degparts, x):
    R = 2000

    def body(dp_ref, x_ref, dis_ref, hp_ref):
        deg = dp_ref[0, :, :16] + dp_ref[1, :, :16] + 1.0
        dis = lax.rsqrt(deg)
        dis_ref[...] = dis
        hp_ref[...] = x_ref[...] * dis[:, 0:1]

    return pl.pallas_call(
        body,
        grid=(N // R,),
        in_specs=[
            pl.BlockSpec((2, R, D), lambda i: (0, i, 0)),
            pl.BlockSpec((R, D), lambda i: (i, 0)),
        ],
        out_specs=[
            pl.BlockSpec((R, 16), lambda i: (i, 0)),
            pl.BlockSpec((R, D), lambda i: (i, 0)),
        ],
        out_shape=[
            jax.ShapeDtypeStruct((N, 16), jnp.float32),
            jax.ShapeDtypeStruct((N, D), jnp.float32),
        ],
    )(degparts, x)


def _dense_call(sparts, hp, x, dis16, w, beta, lin_wT=None, lin_bR=None):
    R = 2000
    last = lin_wT is not None

    def body(sp_ref, hp_ref, x_ref, dis_ref, w_ref, *rest):
        d1 = dis_ref[:, 0:1]
        agg = d1 * (sp_ref[0] + sp_ref[1] + hp_ref[...])
        agg = (1.0 - ALPHA) * agg + ALPHA * x_ref[...]
        out = (1.0 - beta) * agg + beta * jnp.dot(
            agg, w_ref[...], preferred_element_type=jnp.float32,
            precision=lax.Precision.HIGHEST)
        h = jnp.maximum(out, 0.0)
        if last:
            lw_ref, lb_ref, o_ref = rest
            o_ref[...] = jnp.dot(
                h, lw_ref[...], preferred_element_type=jnp.float32,
                precision=lax.Precision.HIGHEST) + lb_ref[...]
        else:
            (o_ref,) = rest
            o_ref[...] = d1 * h

    in_specs = [
        pl.BlockSpec((2, R, D), lambda i: (0, i, 0)),
        pl.BlockSpec((R, D), lambda i: (i, 0)),
        pl.BlockSpec((R, D), lambda i: (i, 0)),
        pl.BlockSpec((R, 16), lambda i: (i, 0)),
        pl.BlockSpec((D, D), lambda i: (0, 0)),
    ]
    args = [sparts, hp, x, dis16, w]
    if last:
        in_specs += [
            pl.BlockSpec((D, OUT), lambda i: (0, 0)),
            pl.BlockSpec((1, OUT), lambda i: (0, 0)),
        ]
        args += [lin_wT, lin_bR]
        out_spec = pl.BlockSpec((R, OUT), lambda i: (i, 0))
        out_shape = jax.ShapeDtypeStruct((N, OUT), jnp.float32)
    else:
        out_spec = pl.BlockSpec((R, D), lambda i: (i, 0))
        out_shape = jax.ShapeDtypeStruct((N, D), jnp.float32)

    return pl.pallas_call(
        body,
        grid=(N // R,),
        in_specs=in_specs,
        out_specs=out_spec,
        out_shape=out_shape,
    )(*args)


def kernel(x, edge_index, edge_weights, conv_w, lin_w, lin_b):
    row4 = edge_index[0].reshape(NW, KOB, G, B)
    col4 = edge_index[1].reshape(NW, KOB, G, B)
    wrep = jnp.broadcast_to(edge_weights[:, None], (E, 16))
    w4d = wrep.reshape(NW, CH, B, 16)
    z128 = jnp.zeros((NPT, D), jnp.float32)

    degparts = _seg_sum_kernel(D, True)(col4, w4d, z128).reshape(2, NPAD, D)
    dis16, hp = _prep_call(degparts, x)

    lin_wT = lin_w.T
    lin_bR = lin_b.reshape(1, OUT)

    seg = _seg_sum_kernel(D, False)
    for l in range(NUM_LAYERS):
        sparts = seg(hp, row4, col4, w4d, z128).reshape(2, NPAD, D)
        beta = math.log(THETA / (l + 1) + 1.0)
        if l < NUM_LAYERS - 1:
            hp = _dense_call(sparts, hp, x, dis16, conv_w[l], beta)
        else:
            out = _dense_call(sparts, hp, x, dis16, conv_w[l], beta,
                              lin_wT, lin_bR)
    return out
